# baseline (speedup 1.0000x reference)
import functools

import jax
import jax.numpy as jnp
from jax.experimental import pallas as pl
from jax.experimental.pallas import tpu as pltpu

TEMPERATURE = 1.0
NBUF = 3


def _body(w1_ref, b1_ref, w2_ref, b2_ref, gl_ref, qb_ref,
          hid_any, qw_any, mem_any,
          out_any, ids_ref, w_ref,
          inb, outb, hsc, acc_s, qwsc, qwc_s, memc_s, memc_bf,
          bias_s, wexp_s, in_sem, out_sem, gsem,
          *, B, S, D, K, MEM, NUM_SLOTS, TS, NST):
    i = pl.program_id(0)
    KM = K * MEM
    NF = B * NST

    def fetch(t):
        bb = t // NST
        st = jax.lax.rem(t, NST)
        slot = jax.lax.rem(t, NBUF)
        return pltpu.make_async_copy(
            hid_any.at[bb, pl.ds(st * TS, TS), :],
            inb.at[slot], in_sem.at[slot])

    @pl.when(i == 0)
    def _prime():
        for t in range(min(NBUF, NF)):
            pltpu.make_async_copy(
                hid_any.at[t // NST, pl.ds((t % NST) * TS, TS), :],
                inb.at[t % NBUF], in_sem.at[t % NBUF]).start()

    @pl.when((i >= 1) & (i + NBUF - 1 < NF))
    def _prefetch():
        fetch(i + NBUF - 1).start()

    @pl.when(i < NF)
    def _fill():
        bb = i // NST
        st = jax.lax.rem(i, NST)
        par = jax.lax.rem(bb, 2)
        fetch(i).wait()
        h_t = inb[jax.lax.rem(i, NBUF)]
        hsc[par, pl.ds(st * TS, TS), :] = h_t.astype(jnp.bfloat16)
        psum = jnp.sum(h_t, axis=0, keepdims=True)

        @pl.when(st == 0)
        def _():
            acc_s[...] = psum

        @pl.when(st > 0)
        def _():
            acc_s[...] += psum

        @pl.when(st == NST - 1)
        def _route():
            q = acc_s[...] * (1.0 / S)
            hmlp = jnp.maximum(
                jnp.dot(q, w1_ref[...], preferred_element_type=jnp.float32)
                + b1_ref[...], 0.0)
            logits = (jnp.dot(hmlp, w2_ref[...],
                              preferred_element_type=jnp.float32)
                      + b2_ref[...]) / TEMPERATURE
            iota_ns = jax.lax.broadcasted_iota(jnp.int32, (1, NUM_SLOTS), 1)
            l = logits
            m_list, idx_list = [], []
            for _ in range(K):
                m = jnp.max(l)
                idx = jnp.min(jnp.where(l == m, iota_ns, NUM_SLOTS))
                m_list.append(m)
                idx_list.append(idx)
                l = jnp.where(iota_ns == idx, -1e30, l)
            for j in range(K):
                pltpu.make_async_copy(
                    qw_any.at[idx_list[j]], qwsc.at[j], gsem).start()
                pltpu.make_async_copy(
                    mem_any.at[idx_list[j]],
                    memc_s.at[par, pl.ds(j * MEM, MEM), :], gsem).start()
            e_list = [jnp.exp(m - m_list[0]) for m in m_list]
            esum = e_list[0]
            for e in e_list[1:]:
                esum = esum + e
            w_list = [e / esum for e in e_list]
            ck = jax.lax.broadcasted_iota(jnp.int32, (1, 1, K), 2)
            ids_out = jnp.zeros((1, 1, K), jnp.int32)
            w_out = jnp.zeros((1, 1, K), jnp.float32)
            for j in range(K):
                ids_out = jnp.where(ck == j, idx_list[j], ids_out)
                w_out = jnp.where(ck == j, w_list[j], w_out)
            ids_ref[...] = ids_out
            w_ref[...] = w_out
            tbl = gl_ref[...] + qb_ref[...]
            colc = jax.lax.broadcasted_iota(jnp.int32, (MEM, KM), 1)
            rowc = jax.lax.broadcasted_iota(jnp.int32, (MEM, KM), 0)
            bias_out = jnp.zeros((1, KM), jnp.float32)
            wexp_out = jnp.zeros((1, KM), jnp.float32)
            ckm = jax.lax.broadcasted_iota(jnp.int32, (1, KM), 1) // MEM
            for j in range(K):
                oh = (iota_ns == idx_list[j]).astype(jnp.float32)
                bj = jnp.dot(oh, tbl, preferred_element_type=jnp.float32)
                selc = (colc == rowc + j * MEM).astype(jnp.float32)
                bias_out = bias_out + jnp.dot(
                    bj, selc, preferred_element_type=jnp.float32)
                wexp_out = jnp.where(ckm == j, w_list[j], wexp_out)
            bias_s[par] = bias_out
            wexp_s[par] = wexp_out

    c = i - NST

    @pl.when(c >= 0)
    def _compute():
        bb = c // NST
        st = jax.lax.rem(c, NST)
        pac = jax.lax.rem(bb, 2)
        slot = jax.lax.rem(c, NBUF)

        @pl.when(st == 0)
        def _pack():
            for j in range(K):
                pltpu.make_async_copy(qw_any.at[0], qwsc.at[j], gsem).wait()
                pltpu.make_async_copy(
                    mem_any.at[0],
                    memc_s.at[pac, pl.ds(j * MEM, MEM), :], gsem).wait()
            colc = jax.lax.broadcasted_iota(jnp.int32, (MEM, KM), 1)
            rowc = jax.lax.broadcasted_iota(jnp.int32, (MEM, KM), 0)
            qwc_s[pac] = jnp.zeros((D, KM), jnp.bfloat16)
            for j in range(K):
                selc = (colc == rowc + j * MEM).astype(jnp.bfloat16)
                qwc_s[pac] += jnp.dot(qwsc[j].astype(jnp.bfloat16), selc,
                                      preferred_element_type=jnp.float32
                                      ).astype(jnp.bfloat16)
            memc_bf[pac] = memc_s[pac].astype(jnp.bfloat16)

        @pl.when(c >= NBUF)
        def _war():
            pltpu.make_async_copy(
                outb.at[slot], out_any.at[0, pl.ds(0, TS), :],
                out_sem.at[slot]).wait()

        h_bf = hsc[pac, pl.ds(st * TS, TS), :]
        scores = jnp.dot(h_bf, qwc_s[pac],
                         preferred_element_type=jnp.float32)
        g = jax.nn.sigmoid(scores + bias_s[pac]) * wexp_s[pac]
        outb[slot] = (h_bf.astype(jnp.float32)
                      + jnp.dot(g.astype(jnp.bfloat16), memc_bf[pac],
                                preferred_element_type=jnp.float32))
        pltpu.make_async_copy(
            outb.at[slot], out_any.at[bb, pl.ds(st * TS, TS), :],
            out_sem.at[slot]).start()

    @pl.when(i == NF + NST - 1)
    def _drain():
        for d in range(min(NBUF, B * NST)):
            slot = (B * NST - 1 - d) % NBUF
            pltpu.make_async_copy(
                outb.at[slot], out_any.at[0, pl.ds(0, TS), :],
                out_sem.at[slot]).wait()


def kernel(hidden_states, W1, b1, W2, b2, memory, gate_logits, qW, qb, top_k):
    B, S, D = hidden_states.shape
    NUM_SLOTS, MEM, _ = memory.shape
    RH = W1.shape[1]
    K = 8
    KM = K * MEM
    TS = 256
    NST = S // TS
    NT = B * NST + NST

    fused = pl.pallas_call(
        functools.partial(_body, B=B, S=S, D=D, K=K, MEM=MEM,
                          NUM_SLOTS=NUM_SLOTS, TS=TS, NST=NST),
        grid=(NT,),
        in_specs=[
            pl.BlockSpec((D, RH), lambda i: (0, 0)),
            pl.BlockSpec((1, RH), lambda i: (0, 0)),
            pl.BlockSpec((RH, NUM_SLOTS), lambda i: (0, 0)),
            pl.BlockSpec((1, NUM_SLOTS), lambda i: (0, 0)),
            pl.BlockSpec((NUM_SLOTS, MEM), lambda i: (0, 0)),
            pl.BlockSpec((NUM_SLOTS, MEM), lambda i: (0, 0)),
            pl.BlockSpec(memory_space=pl.ANY),
            pl.BlockSpec(memory_space=pl.ANY),
            pl.BlockSpec(memory_space=pl.ANY),
        ],
        out_specs=[
            pl.BlockSpec(memory_space=pl.ANY),
            pl.BlockSpec((1, 1, K),
                         lambda i: (jnp.minimum(i // NST, B - 1), 0, 0)),
            pl.BlockSpec((1, 1, K),
                         lambda i: (jnp.minimum(i // NST, B - 1), 0, 0)),
        ],
        out_shape=[
            jax.ShapeDtypeStruct((B, S, D), jnp.float32),
            jax.ShapeDtypeStruct((B, 1, K), jnp.int32),
            jax.ShapeDtypeStruct((B, 1, K), jnp.float32),
        ],
        scratch_shapes=[
            pltpu.VMEM((NBUF, TS, D), jnp.float32),
            pltpu.VMEM((NBUF, TS, D), jnp.float32),
            pltpu.VMEM((2, S, D), jnp.bfloat16),
            pltpu.VMEM((1, D), jnp.float32),
            pltpu.VMEM((K, D, MEM), jnp.float32),
            pltpu.VMEM((2, D, KM), jnp.bfloat16),
            pltpu.VMEM((2, KM, D), jnp.float32),
            pltpu.VMEM((2, KM, D), jnp.bfloat16),
            pltpu.VMEM((2, 1, KM), jnp.float32),
            pltpu.VMEM((2, 1, KM), jnp.float32),
            pltpu.SemaphoreType.DMA((NBUF,)),
            pltpu.SemaphoreType.DMA((NBUF,)),
            pltpu.SemaphoreType.DMA,
        ],
        compiler_params=pltpu.CompilerParams(
            dimension_semantics=("arbitrary",)),
    )
    out, ids3, w3 = fused(
        W1, b1.reshape(1, RH), W2, b2.reshape(1, NUM_SLOTS),
        gate_logits, qb, hidden_states, qW, memory)
    return out, ids3.reshape(B, K), w3.reshape(B, K)


# speedup vs baseline: 1.0090x; 1.0090x over previous
import functools

import jax
import jax.numpy as jnp
from jax.experimental import pallas as pl
from jax.experimental.pallas import tpu as pltpu

TEMPERATURE = 1.0
NBUF = 4


def _body(w1_ref, b1_ref, w2_ref, b2_ref, gl_ref, qb_ref,
          hid_any, qw_any, mem_any,
          out_any, ids_ref, w_ref,
          inb, outb, hsc, acc_s, qwsc, qwc_s, memc_s, memc_bf,
          bias_s, wexp_s, in_sem, out_sem, gsem,
          *, B, S, D, K, MEM, NUM_SLOTS, TS, NST):
    i = pl.program_id(0)
    KM = K * MEM
    NF = B * NST

    def fetch(t):
        bb = t // NST
        st = jax.lax.rem(t, NST)
        slot = jax.lax.rem(t, NBUF)
        return pltpu.make_async_copy(
            hid_any.at[bb, pl.ds(st * TS, TS), :],
            inb.at[slot], in_sem.at[slot])

    @pl.when(i == 0)
    def _prime():
        for t in range(min(NBUF, NF)):
            pltpu.make_async_copy(
                hid_any.at[t // NST, pl.ds((t % NST) * TS, TS), :],
                inb.at[t % NBUF], in_sem.at[t % NBUF]).start()

    @pl.when((i >= 1) & (i + NBUF - 1 < NF))
    def _prefetch():
        fetch(i + NBUF - 1).start()

    @pl.when(i < NF)
    def _fill():
        bb = i // NST
        st = jax.lax.rem(i, NST)
        par = jax.lax.rem(bb, 2)
        fetch(i).wait()
        h_t = inb[jax.lax.rem(i, NBUF)]
        hsc[par, pl.ds(st * TS, TS), :] = h_t.astype(jnp.bfloat16)
        psum = jnp.sum(h_t, axis=0, keepdims=True)

        @pl.when(st == 0)
        def _():
            acc_s[...] = psum

        @pl.when(st > 0)
        def _():
            acc_s[...] += psum

        @pl.when(st == NST - 1)
        def _route():
            q = acc_s[...] * (1.0 / S)
            hmlp = jnp.maximum(
                jnp.dot(q, w1_ref[...], preferred_element_type=jnp.float32)
                + b1_ref[...], 0.0)
            logits = (jnp.dot(hmlp, w2_ref[...],
                              preferred_element_type=jnp.float32)
                      + b2_ref[...]) / TEMPERATURE
            iota_ns = jax.lax.broadcasted_iota(jnp.int32, (1, NUM_SLOTS), 1)
            l = logits
            m_list, idx_list = [], []
            for _ in range(K):
                m = jnp.max(l)
                idx = jnp.min(jnp.where(l == m, iota_ns, NUM_SLOTS))
                m_list.append(m)
                idx_list.append(idx)
                l = jnp.where(iota_ns == idx, -1e30, l)
            for j in range(K):
                pltpu.make_async_copy(
                    qw_any.at[idx_list[j]], qwsc.at[j], gsem).start()
                pltpu.make_async_copy(
                    mem_any.at[idx_list[j]],
                    memc_s.at[par, pl.ds(j * MEM, MEM), :], gsem).start()
            e_list = [jnp.exp(m - m_list[0]) for m in m_list]
            esum = e_list[0]
            for e in e_list[1:]:
                esum = esum + e
            w_list = [e / esum for e in e_list]
            ck = jax.lax.broadcasted_iota(jnp.int32, (1, 1, K), 2)
            ids_out = jnp.zeros((1, 1, K), jnp.int32)
            w_out = jnp.zeros((1, 1, K), jnp.float32)
            for j in range(K):
                ids_out = jnp.where(ck == j, idx_list[j], ids_out)
                w_out = jnp.where(ck == j, w_list[j], w_out)
            ids_ref[...] = ids_out
            w_ref[...] = w_out
            tbl = gl_ref[...] + qb_ref[...]
            colc = jax.lax.broadcasted_iota(jnp.int32, (MEM, KM), 1)
            rowc = jax.lax.broadcasted_iota(jnp.int32, (MEM, KM), 0)
            bias_out = jnp.zeros((1, KM), jnp.float32)
            wexp_out = jnp.zeros((1, KM), jnp.float32)
            ckm = jax.lax.broadcasted_iota(jnp.int32, (1, KM), 1) // MEM
            for j in range(K):
                oh = (iota_ns == idx_list[j]).astype(jnp.float32)
                bj = jnp.dot(oh, tbl, preferred_element_type=jnp.float32)
                selc = (colc == rowc + j * MEM).astype(jnp.float32)
                bias_out = bias_out + jnp.dot(
                    bj, selc, preferred_element_type=jnp.float32)
                wexp_out = jnp.where(ckm == j, w_list[j], wexp_out)
            bias_s[par] = bias_out
            wexp_s[par] = wexp_out

    c = i - NST

    @pl.when(c >= 0)
    def _compute():
        bb = c // NST
        st = jax.lax.rem(c, NST)
        pac = jax.lax.rem(bb, 2)
        slot = jax.lax.rem(c, NBUF)

        @pl.when(st == 0)
        def _pack():
            for j in range(K):
                pltpu.make_async_copy(qw_any.at[0], qwsc.at[j], gsem).wait()
                pltpu.make_async_copy(
                    mem_any.at[0],
                    memc_s.at[pac, pl.ds(j * MEM, MEM), :], gsem).wait()
            colc = jax.lax.broadcasted_iota(jnp.int32, (MEM, KM), 1)
            rowc = jax.lax.broadcasted_iota(jnp.int32, (MEM, KM), 0)
            qwc_s[pac] = jnp.zeros((D, KM), jnp.bfloat16)
            for j in range(K):
                selc = (colc == rowc + j * MEM).astype(jnp.bfloat16)
                qwc_s[pac] += jnp.dot(qwsc[j].astype(jnp.bfloat16), selc,
                                      preferred_element_type=jnp.float32
                                      ).astype(jnp.bfloat16)
            memc_bf[pac] = memc_s[pac].astype(jnp.bfloat16)

        @pl.when(c >= NBUF)
        def _war():
            pltpu.make_async_copy(
                outb.at[slot], out_any.at[0, pl.ds(0, TS), :],
                out_sem.at[slot]).wait()

        h_bf = hsc[pac, pl.ds(st * TS, TS), :]
        scores = jnp.dot(h_bf, qwc_s[pac],
                         preferred_element_type=jnp.float32)
        g = jax.nn.sigmoid(scores + bias_s[pac]) * wexp_s[pac]
        outb[slot] = (h_bf.astype(jnp.float32)
                      + jnp.dot(g.astype(jnp.bfloat16), memc_bf[pac],
                                preferred_element_type=jnp.float32))
        pltpu.make_async_copy(
            outb.at[slot], out_any.at[bb, pl.ds(st * TS, TS), :],
            out_sem.at[slot]).start()

    @pl.when(i == NF + NST - 1)
    def _drain():
        for d in range(min(NBUF, B * NST)):
            slot = (B * NST - 1 - d) % NBUF
            pltpu.make_async_copy(
                outb.at[slot], out_any.at[0, pl.ds(0, TS), :],
                out_sem.at[slot]).wait()


def kernel(hidden_states, W1, b1, W2, b2, memory, gate_logits, qW, qb, top_k):
    B, S, D = hidden_states.shape
    NUM_SLOTS, MEM, _ = memory.shape
    RH = W1.shape[1]
    K = 8
    KM = K * MEM
    TS = 256
    NST = S // TS
    NT = B * NST + NST

    fused = pl.pallas_call(
        functools.partial(_body, B=B, S=S, D=D, K=K, MEM=MEM,
                          NUM_SLOTS=NUM_SLOTS, TS=TS, NST=NST),
        grid=(NT,),
        in_specs=[
            pl.BlockSpec((D, RH), lambda i: (0, 0)),
            pl.BlockSpec((1, RH), lambda i: (0, 0)),
            pl.BlockSpec((RH, NUM_SLOTS), lambda i: (0, 0)),
            pl.BlockSpec((1, NUM_SLOTS), lambda i: (0, 0)),
            pl.BlockSpec((NUM_SLOTS, MEM), lambda i: (0, 0)),
            pl.BlockSpec((NUM_SLOTS, MEM), lambda i: (0, 0)),
            pl.BlockSpec(memory_space=pl.ANY),
            pl.BlockSpec(memory_space=pl.ANY),
            pl.BlockSpec(memory_space=pl.ANY),
        ],
        out_specs=[
            pl.BlockSpec(memory_space=pl.ANY),
            pl.BlockSpec((1, 1, K),
                         lambda i: (jnp.minimum(i // NST, B - 1), 0, 0)),
            pl.BlockSpec((1, 1, K),
                         lambda i: (jnp.minimum(i // NST, B - 1), 0, 0)),
        ],
        out_shape=[
            jax.ShapeDtypeStruct((B, S, D), jnp.float32),
            jax.ShapeDtypeStruct((B, 1, K), jnp.int32),
            jax.ShapeDtypeStruct((B, 1, K), jnp.float32),
        ],
        scratch_shapes=[
            pltpu.VMEM((NBUF, TS, D), jnp.float32),
            pltpu.VMEM((NBUF, TS, D), jnp.float32),
            pltpu.VMEM((2, S, D), jnp.bfloat16),
            pltpu.VMEM((1, D), jnp.float32),
            pltpu.VMEM((K, D, MEM), jnp.float32),
            pltpu.VMEM((2, D, KM), jnp.bfloat16),
            pltpu.VMEM((2, KM, D), jnp.float32),
            pltpu.VMEM((2, KM, D), jnp.bfloat16),
            pltpu.VMEM((2, 1, KM), jnp.float32),
            pltpu.VMEM((2, 1, KM), jnp.float32),
            pltpu.SemaphoreType.DMA((NBUF,)),
            pltpu.SemaphoreType.DMA((NBUF,)),
            pltpu.SemaphoreType.DMA,
        ],
        compiler_params=pltpu.CompilerParams(
            dimension_semantics=("arbitrary",)),
    )
    out, ids3, w3 = fused(
        W1, b1.reshape(1, RH), W2, b2.reshape(1, NUM_SLOTS),
        gate_logits, qb, hidden_states, qW, memory)
    return out, ids3.reshape(B, K), w3.reshape(B, K)
